# Initial kernel scaffold; baseline (speedup 1.0000x reference)
import functools

import jax
import jax.numpy as jnp
from jax import lax
from jax.experimental import pallas as pl
from jax.experimental.pallas import tpu as pltpu
from jax.experimental.pallas import tpu_sc as plsc

BATCH = 4096
SEQ = 200
EMB = 32
LAT = 64
NTOK = BATCH * SEQ
NC, NS = 2, 16
NW = NC * NS
PER_W = NTOK // NW
CHUNK = 512
NBUF = 2
NCHUNK = PER_W // CHUNK
assert NCHUNK % NBUF == 0 and PER_W % CHUNK == 0


def _sc_gather(x_flat, embed_table):
    mesh = plsc.VectorSubcoreMesh(core_axis_name="c", subcore_axis_name="s")

    @functools.partial(
        pl.kernel,
        out_type=jax.ShapeDtypeStruct((NTOK, EMB), jnp.float32),
        mesh=mesh,
        scratch_types=[
            pltpu.VMEM((NBUF, CHUNK), jnp.int32),
            pltpu.VMEM((NBUF, CHUNK, EMB), jnp.float32),
            pltpu.SemaphoreType.DMA,
            pltpu.SemaphoreType.DMA,
        ],
    )
    def gather_kernel(x_hbm, table_hbm, out_hbm, idx_v, rows_v, sem0, sem1):
        wid = lax.axis_index("s") * NC + lax.axis_index("c")
        base = wid * PER_W
        sems = [sem0, sem1]

        def start(c, slot):
            off = base + c * CHUNK
            pltpu.sync_copy(x_hbm.at[pl.ds(off, CHUNK)], idx_v.at[slot])
            pltpu.async_copy(table_hbm.at[idx_v.at[slot]], rows_v.at[slot],
                             sems[slot])

        def drain(c, slot):
            off = base + c * CHUNK
            pltpu.make_async_copy(table_hbm.at[idx_v.at[slot]],
                                  rows_v.at[slot], sems[slot]).wait()
            pltpu.sync_copy(rows_v.at[slot], out_hbm.at[pl.ds(off, CHUNK)])

        for b in range(NBUF):
            start(b, b)

        @pl.loop(0, NCHUNK - NBUF, step=NBUF)
        def _ring(c):
            for b in range(NBUF):
                drain(c + b, b)
                start(c + NBUF + b, b)

        for b in range(NBUF):
            drain(NCHUNK - NBUF + b, b)

    return gather_kernel(x_flat, embed_table)


def _tc_encode(e2d, pos_flat, dense_kernel, bias2d):
    BM = 256
    grid = (BATCH // BM,)

    def mm_kernel(e_ref, pos_ref, w_ref, b_ref, o_ref):
        e = e_ref[...] + pos_ref[...]
        o_ref[...] = (
            jnp.dot(e, w_ref[...], preferred_element_type=jnp.float32)
            + b_ref[...]
        )

    return pl.pallas_call(
        mm_kernel,
        grid=grid,
        in_specs=[
            pl.BlockSpec((BM, SEQ * EMB), lambda i: (i, 0)),
            pl.BlockSpec((1, SEQ * EMB), lambda i: (0, 0)),
            pl.BlockSpec((SEQ * EMB, LAT), lambda i: (0, 0)),
            pl.BlockSpec((1, LAT), lambda i: (0, 0)),
        ],
        out_specs=pl.BlockSpec((BM, LAT), lambda i: (i, 0)),
        out_shape=jax.ShapeDtypeStruct((BATCH, LAT), jnp.float32),
    )(e2d, pos_flat, dense_kernel, bias2d)


def kernel(x, embed_table, pos_emb, dense_kernel, dense_bias):
    x_flat = x.reshape((NTOK,))
    e = _sc_gather(x_flat, embed_table)
    e2d = e.reshape((BATCH, SEQ * EMB))
    pos_flat = pos_emb.reshape((1, SEQ * EMB))
    bias2d = dense_bias.reshape((1, LAT))
    return _tc_encode(e2d, pos_flat, dense_kernel, bias2d)


# speedup vs baseline: 2.0438x; 2.0438x over previous
import functools

import jax
import jax.numpy as jnp
from jax import lax
from jax.experimental import pallas as pl
from jax.experimental.pallas import tpu as pltpu
from jax.experimental.pallas import tpu_sc as plsc

BATCH = 4096
SEQ = 200
EMB = 32
LAT = 64
NTOK = BATCH * SEQ
NC, NS = 2, 16
NW = NC * NS
PER_W = NTOK // NW
CHUNK = 512
NBUF = 2
NCHUNK = PER_W // CHUNK
assert NCHUNK % NBUF == 0 and PER_W % CHUNK == 0


def _sc_gather(x_flat, embed_table):
    mesh = plsc.VectorSubcoreMesh(core_axis_name="c", subcore_axis_name="s")

    @functools.partial(
        pl.kernel,
        out_type=jax.ShapeDtypeStruct((NTOK, EMB), jnp.float32),
        mesh=mesh,
        scratch_types=[
            pltpu.VMEM((NBUF, CHUNK), jnp.int32),
            pltpu.VMEM((NBUF, CHUNK, EMB), jnp.float32),
            pltpu.SemaphoreType.DMA,
            pltpu.SemaphoreType.DMA,
        ],
        compiler_params=pltpu.CompilerParams(use_tc_tiling_on_sc=False),
    )
    def gather_kernel(x_hbm, table_hbm, out_hbm, idx_v, rows_v, sem0, sem1):
        wid = lax.axis_index("s") * NC + lax.axis_index("c")
        base = wid * PER_W
        sems = [sem0, sem1]

        def start(c, slot):
            off = base + c * CHUNK
            pltpu.sync_copy(x_hbm.at[pl.ds(off, CHUNK)], idx_v.at[slot])
            pltpu.async_copy(table_hbm.at[idx_v.at[slot]], rows_v.at[slot],
                             sems[slot])

        def drain(c, slot):
            off = base + c * CHUNK
            pltpu.make_async_copy(table_hbm.at[idx_v.at[slot]],
                                  rows_v.at[slot], sems[slot]).wait()
            pltpu.sync_copy(rows_v.at[slot], out_hbm.at[pl.ds(off, CHUNK)])

        for b in range(NBUF):
            start(b, b)

        @pl.loop(0, NCHUNK - NBUF, step=NBUF)
        def _ring(c):
            for b in range(NBUF):
                drain(c + b, b)
                start(c + NBUF + b, b)

        for b in range(NBUF):
            drain(NCHUNK - NBUF + b, b)

    return gather_kernel(x_flat, embed_table)


def _tc_encode(e2d, pos_flat, dense_kernel, bias2d):
    BM = 256
    grid = (BATCH // BM,)

    def mm_kernel(e_ref, pos_ref, w_ref, b_ref, o_ref):
        e = e_ref[...] + pos_ref[...]
        o_ref[...] = (
            jnp.dot(e, w_ref[...], preferred_element_type=jnp.float32)
            + b_ref[...]
        )

    return pl.pallas_call(
        mm_kernel,
        grid=grid,
        in_specs=[
            pl.BlockSpec((BM, SEQ * EMB), lambda i: (i, 0)),
            pl.BlockSpec((1, SEQ * EMB), lambda i: (0, 0)),
            pl.BlockSpec((SEQ * EMB, LAT), lambda i: (0, 0)),
            pl.BlockSpec((1, LAT), lambda i: (0, 0)),
        ],
        out_specs=pl.BlockSpec((BM, LAT), lambda i: (i, 0)),
        out_shape=jax.ShapeDtypeStruct((BATCH, LAT), jnp.float32),
    )(e2d, pos_flat, dense_kernel, bias2d)


def kernel(x, embed_table, pos_emb, dense_kernel, dense_bias):
    x_flat = x.reshape((NTOK,))
    e = _sc_gather(x_flat, embed_table)
    e2d = e.reshape((BATCH, SEQ * EMB))
    pos_flat = pos_emb.reshape((1, SEQ * EMB))
    bias2d = dense_bias.reshape((1, LAT))
    return _tc_encode(e2d, pos_flat, dense_kernel, bias2d)
